# baseline (speedup 1.0000x reference)
import jax
import jax.numpy as jnp
from jax.experimental import pallas as pl
from jax.experimental.pallas import tpu as pltpu

N = 100000
B = 1024
NB = 10


def _cls_body(x4_ref, wl_ref, bl_ref, wc_ref, bc_ref, o_ref):
    x4 = x4_ref[...]
    hl = jnp.dot(x4, wl_ref[...], preferred_element_type=jnp.float32) + bl_ref[...]
    out = (jnp.dot(hl, wc_ref[0:64, :], preferred_element_type=jnp.float32)
           + jnp.dot(x4, wc_ref[64:82, :], preferred_element_type=jnp.float32)
           + bc_ref[...])
    o_ref[...] = out


def kernel(x, edge_index, edge_attr, batch, Wf1, bf1, Ws1, bs1, Wf2, bf2, Ws2, bs2, Wl, bl, Wc, bc):
    def cgconv(h, Wf, bf, Ws, bs):
        src = edge_index[0]
        dst = edge_index[1]
        z = jnp.concatenate([h[dst], h[src], edge_attr], axis=-1)
        m = jax.nn.sigmoid(z @ Wf + bf) * jax.nn.softplus(z @ Ws + bs)
        return h + jax.ops.segment_sum(m, dst, num_segments=h.shape[0])

    def pools(h):
        mx = jax.ops.segment_max(h, batch, num_segments=B)
        sm = jax.ops.segment_sum(h, batch, num_segments=B)
        cnt = jax.ops.segment_sum(jnp.ones((h.shape[0], 1), h.dtype), batch, num_segments=B)
        return jnp.concatenate([mx, sm / jnp.maximum(cnt, 1.0)], axis=1)

    h = cgconv(x, Wf1, bf1, Ws1, bs1)
    x1 = pools(h)
    h = cgconv(h, Wf2, bf2, Ws2, bs2)
    x2 = pools(h)
    h = cgconv(h, Wf2, bf2, Ws2, bs2)
    x3 = pools(h)
    x4 = jnp.concatenate([x1, x2, x3], axis=1)

    out = pl.pallas_call(
        _cls_body,
        out_shape=jax.ShapeDtypeStruct((B, NB), jnp.float32),
    )(x4, Wl, bl, Wc, bc)
    return out


# speedup vs baseline: 25.7204x; 25.7204x over previous
import functools

import jax
import jax.numpy as jnp
from jax import lax
from jax.experimental import pallas as pl
from jax.experimental.pallas import tpu as pltpu
from jax.experimental.pallas import tpu_sc as plsc

N = 100000
E = 3200000
B = 1024
NC_OUT = 10

NP = 100352
ROWS = E // 128
RPAD = 25088
EP = RPAD * 128
W_ALL = 49
NPW = NP // 32
HCH = NPW // 2
SL3 = NP * 3 // 16
BIG = 3.4e38

_mesh = plsc.VectorSubcoreMesh(
    core_axis_name="c", subcore_axis_name="s", num_cores=2, num_subcores=16)
_params = pltpu.CompilerParams(needs_layout_passes=False)


def _sigmoid(f):
    return 1.0 / (1.0 + jnp.exp(-f))


def _softplus(s):
    u = jnp.exp(-jnp.abs(s))
    t = u / (2.0 + u)
    t2 = t * t
    p = 1.0 + t2 * (0.33333333 + t2 * (0.2 + t2 * (0.14285714 + t2 * 0.11111111)))
    return jnp.maximum(s, 0.0) + 2.0 * t * p


def _bf16r(x):
    u = plsc.bitcast(x, jnp.int32)
    r = (u + 0x7FFF + ((u >> 16) & 1)) & (-65536)
    return plsc.bitcast(r, jnp.float32)


def _edge_body(xp, srcf, dstf, eat, wvec, agg0, agg1,
               xsh, acc, sidx, didx, sg4, dg4, eacv, xdv, xsv, mv, bnc,
               wsm, gsem, ssem):
    cid = lax.axis_index("c")
    sid = lax.axis_index("s")
    w = cid * 16 + sid
    lo = w * (RPAD // 32)

    iota = lax.iota(jnp.int32, 16)
    zero16 = jnp.zeros((16,), jnp.float32)

    def zb(i, carry):
        bnc[pl.ds(i * 16, 16)] = zero16
        return carry

    lax.fori_loop(0, 6272 // 16, zb, 0)
    pltpu.sync_copy(xp.at[pl.ds(sid * SL3, SL3)], xsh.at[pl.ds(sid * SL3, SL3)])
    for q in range(3):
        pltpu.sync_copy(bnc, acc.at[pl.ds(sid * SL3 + q * 6272, 6272)])

    pltpu.sync_copy(wvec, wsm)
    plsc.subcore_barrier()

    def process_window(row0):
        base = row0 * 128
        pltpu.sync_copy(srcf.at[pl.ds(base, 2048)], sidx)
        pltpu.sync_copy(dstf.at[pl.ds(base, 2048)], didx)
        for i in range(4):
            pltpu.sync_copy(eat.at[pl.ds(i * EP + base, 2048)],
                            eacv.at[pl.ds(i * 2048, 2048)])

        def build(g, carry):
            ch = g >> 3
            cs = (g & 7) * 16
            sv3 = sidx[pl.ds(g * 16, 16)] * 3
            dv3 = didx[pl.ds(g * 16, 16)] * 3
            for c in range(3):
                sg4[c * 16 + ch, pl.ds(cs, 16)] = sv3 + c
                dg4[c * 16 + ch, pl.ds(cs, 16)] = dv3 + c
            return carry

        lax.fori_loop(0, 128, build, 0)

        descs = []
        for j in range(48):
            descs.append(pltpu.async_copy(
                xsh.at[sg4.at[j]], xsv.at[j], gsem))
            descs.append(pltpu.async_copy(
                xsh.at[dg4.at[j]], xdv.at[j], gsem))
        for d in descs:
            d.wait()

        def group(g, carry):
            ch = g >> 3
            cs = (g & 7) * 16
            xd = [_bf16r(xdv[c * 16 + ch, pl.ds(cs, 16)]) for c in range(3)]
            xs = [_bf16r(xsv[c * 16 + ch, pl.ds(cs, 16)]) for c in range(3)]
            ez = [_bf16r(eacv[pl.ds(i * 2048 + g * 16, 16)]) for i in range(4)]
            for o in range(3):
                f = wsm[60 + o, pl.ds(0, 16)]
                s = wsm[63 + o, pl.ds(0, 16)]
                for i in range(3):
                    f = f + xd[i] * wsm[o * 10 + i, pl.ds(0, 16)]
                    s = s + xd[i] * wsm[30 + o * 10 + i, pl.ds(0, 16)]
                for i in range(3):
                    f = f + xs[i] * wsm[o * 10 + 3 + i, pl.ds(0, 16)]
                    s = s + xs[i] * wsm[30 + o * 10 + 3 + i, pl.ds(0, 16)]
                for i in range(4):
                    f = f + ez[i] * wsm[o * 10 + 6 + i, pl.ds(0, 16)]
                    s = s + ez[i] * wsm[30 + o * 10 + 6 + i, pl.ds(0, 16)]
                m = _sigmoid(f) * _softplus(s)
                mv[o * 16 + ch, pl.ds(cs, 16)] = m
            return carry

        lax.fori_loop(0, 128, group, 0)

        sdescs = []
        for j in range(48):
            sdescs.append(pltpu.async_copy(
                mv.at[j], acc.at[dg4.at[j]], ssem, add=True))
        for d in sdescs:
            d.wait()

    def win_body(k, carry):
        process_window(lo + k * 16)
        return carry

    lax.fori_loop(0, W_ALL, win_body, 0)

    plsc.subcore_barrier()

    for q in range(3):
        off = sid * SL3 + q * 6272
        pltpu.sync_copy(acc.at[pl.ds(off, 6272)], bnc)

        @pl.when(cid == 0)
        def _():
            pltpu.sync_copy(bnc, agg0.at[pl.ds(off, 6272)])

        @pl.when(cid == 1)
        def _():
            pltpu.sync_copy(bnc, agg1.at[pl.ds(off, 6272)])


_nf = jax.ShapeDtypeStruct((NP * 3,), jnp.float32)
_edge_kernel = functools.partial(
    pl.kernel,
    out_type=(_nf, _nf),
    mesh=_mesh,
    compiler_params=_params,
    scratch_types=[
        pltpu.VMEM_SHARED((NP * 3,), jnp.float32),
        pltpu.VMEM_SHARED((NP * 3,), jnp.float32),
        pltpu.VMEM((2048,), jnp.int32),
        pltpu.VMEM((2048,), jnp.int32),
        pltpu.VMEM((48, 128), jnp.int32),
        pltpu.VMEM((48, 128), jnp.int32),
        pltpu.VMEM((8192,), jnp.float32),
        pltpu.VMEM((48, 128), jnp.float32),
        pltpu.VMEM((48, 128), jnp.float32),
        pltpu.VMEM((48, 128), jnp.float32),
        pltpu.VMEM((6272,), jnp.float32),
        pltpu.VMEM((72, 16), jnp.float32),
        pltpu.SemaphoreType.DMA,
        pltpu.SemaphoreType.DMA,
    ],
)(_edge_body)


def _pool_body(xp, a0, a1, batchp, hout, pp0, pp1,
               psh, xv, a0v, a1v, hv, bidv, lacc, stg, redv):
    cid = lax.axis_index("c")
    sid = lax.axis_index("s")
    w = cid * 16 + sid
    n0 = w * NPW

    iota = lax.iota(jnp.int32, 16)
    patmax = (iota % 8) < 3
    initv = jnp.where(patmax, -BIG, 0.0)

    def initb(i, carry):
        lacc[pl.ds(i * 16, 16)] = initv
        return carry

    lax.fori_loop(0, B * 8 // 16, initb, 0)

    for chk in range(2):
        c0 = n0 + chk * HCH
        pltpu.sync_copy(xp.at[pl.ds(c0 * 3, HCH * 3)], xv)
        pltpu.sync_copy(a0.at[pl.ds(c0 * 3, HCH * 3)], a0v)
        pltpu.sync_copy(a1.at[pl.ds(c0 * 3, HCH * 3)], a1v)
        pltpu.sync_copy(batchp.at[pl.ds(c0, HCH)], bidv)

        def group(g, carry):
            gi = g * 16 + iota
            gi3 = gi * 3
            hcols = []
            for c in range(3):
                hc = (plsc.load_gather(xv, [gi3 + c])
                      + plsc.load_gather(a0v, [gi3 + c])
                      + plsc.load_gather(a1v, [gi3 + c]))
                plsc.store_scatter(hv, [gi3 + c], hc)
                hcols.append(hc)

            bid = bidv[pl.ds(g * 16, 16)]
            active = bid < B
            mx = [jnp.where(active, hcols[c], -BIG) for c in range(3)]
            sm = [jnp.where(active, hcols[c], 0.0) for c in range(3)]
            cnt = jnp.where(active, 1.0, 0.0)

            for k in (1, 2, 4, 8):
                validk = iota >= k
                sbid = plsc.load_gather(bidv, [jnp.maximum(gi - k, 0)])
                same = jnp.logical_and(sbid == bid, validk)
                lidx = jnp.maximum(iota - k, 0)
                stage = [mx[0], mx[1], mx[2], sm[0], sm[1], sm[2], cnt]
                for a in range(7):
                    stg[pl.ds(a * 16, 16)] = stage[a]
                shifted = [plsc.load_gather(stg, [lidx + a * 16])
                           for a in range(7)]
                for a in range(3):
                    mx[a] = jnp.maximum(mx[a], jnp.where(same, shifted[a], -BIG))
                for a in range(3):
                    sm[a] = sm[a] + jnp.where(same, shifted[3 + a], 0.0)
                cnt = cnt + jnp.where(same, shifted[6], 0.0)

            nbid = plsc.load_gather(bidv, [jnp.minimum(gi + 1, HCH - 1)])
            is_last = jnp.logical_and(
                jnp.logical_or(nbid != bid, iota == 15), active)

            addr0 = bid * 8
            vals = [mx[0], mx[1], mx[2], sm[0], sm[1], sm[2], cnt]
            for a in range(7):
                ad = addr0 + a
                cur = plsc.load_gather(lacc, [ad])
                if a < 3:
                    newv = jnp.maximum(cur, vals[a])
                else:
                    newv = cur + vals[a]
                plsc.store_scatter(lacc, [ad], newv, mask=is_last)
            return carry

        lax.fori_loop(0, HCH // 16, group, 0)
        pltpu.sync_copy(hv, hout.at[pl.ds(c0 * 3, HCH * 3)])

    pltpu.sync_copy(lacc, psh.at[sid])
    plsc.subcore_barrier()
    pltpu.sync_copy(psh.at[:, pl.ds(sid * 512, 512)], redv)

    def redb(i, carry):
        acc = redv[0, pl.ds(i * 16, 16)]
        for t in range(1, 16):
            v = redv[t, pl.ds(i * 16, 16)]
            acc = jnp.where(patmax, jnp.maximum(acc, v), acc + v)
        lacc[pl.ds(i * 16, 16)] = acc
        return carry

    lax.fori_loop(0, 512 // 16, redb, 0)

    @pl.when(cid == 0)
    def _():
        pltpu.sync_copy(lacc.at[pl.ds(0, 512)], pp0.at[pl.ds(sid * 512, 512)])

    @pl.when(cid == 1)
    def _():
        pltpu.sync_copy(lacc.at[pl.ds(0, 512)], pp1.at[pl.ds(sid * 512, 512)])


_pp = jax.ShapeDtypeStruct((B * 8,), jnp.float32)
_pool_kernel = functools.partial(
    pl.kernel,
    out_type=(_nf, _pp, _pp),
    mesh=_mesh,
    compiler_params=_params,
    scratch_types=[
        pltpu.VMEM_SHARED((16, B * 8), jnp.float32),
        pltpu.VMEM((HCH * 3,), jnp.float32),
        pltpu.VMEM((HCH * 3,), jnp.float32),
        pltpu.VMEM((HCH * 3,), jnp.float32),
        pltpu.VMEM((HCH * 3,), jnp.float32),
        pltpu.VMEM((HCH,), jnp.int32),
        pltpu.VMEM((B * 8,), jnp.float32),
        pltpu.VMEM((112,), jnp.float32),
        pltpu.VMEM((16, 512), jnp.float32),
    ],
)(_pool_body)


def _cls_body(p01, p11, p02, p12, p03, p13, wl_ref, bl_ref, wc_ref, bc_ref, o_ref):
    def pool_pair(pa, pb):
        mx = jnp.maximum(pa[:, 0:3], pb[:, 0:3])
        sm = pa[:, 3:6] + pb[:, 3:6]
        cnt = pa[:, 6:7] + pb[:, 6:7]
        return mx, sm / jnp.maximum(cnt, 1.0)

    mx1, mn1 = pool_pair(p01[...], p11[...])
    mx2, mn2 = pool_pair(p02[...], p12[...])
    mx3, mn3 = pool_pair(p03[...], p13[...])
    x4 = jnp.concatenate([mx1, mn1, mx2, mn2, mx3, mn3], axis=1)
    hl = jnp.dot(x4, wl_ref[...], preferred_element_type=jnp.float32) + bl_ref[...]
    out = (jnp.dot(hl, wc_ref[0:64, :], preferred_element_type=jnp.float32)
           + jnp.dot(x4, wc_ref[64:82, :], preferred_element_type=jnp.float32)
           + bc_ref[...])
    o_ref[...] = out


def _bf16r_host(w):
    u = jax.lax.bitcast_convert_type(w, jnp.int32)
    r = (u + 0x7FFF + ((u >> 16) & 1)) & (-65536)
    return jax.lax.bitcast_convert_type(r, jnp.float32)


def _pack_w(Wf, bf, Ws, bs):
    wf = _bf16r_host(Wf)
    ws = _bf16r_host(Ws)
    flat = jnp.concatenate([
        wf.T.reshape(-1), ws.T.reshape(-1), bf, bs,
        jnp.zeros((6,), jnp.float32)])
    return jnp.tile(flat[:, None], (1, 16))


def kernel(x, edge_index, edge_attr, batch, Wf1, bf1, Ws1, bs1, Wf2, bf2, Ws2, bs2, Wl, bl, Wc, bc):
    xp = jnp.zeros((NP, 3), jnp.float32).at[:N].set(x).reshape(-1)
    pad_idx = jnp.full((EP - E,), NP - 1, jnp.int32)
    srcf = jnp.concatenate([edge_index[0], pad_idx])
    dstf = jnp.concatenate([edge_index[1], pad_idx])
    eat = jnp.concatenate(
        [edge_attr, jnp.zeros((EP - E, 4), jnp.float32)]).T.reshape(-1)
    batchp = jnp.concatenate([batch, jnp.full((NP - N,), B, jnp.int32)])
    w1 = _pack_w(Wf1, bf1, Ws1, bs1)
    w2 = _pack_w(Wf2, bf2, Ws2, bs2)

    a0, a1 = _edge_kernel(xp, srcf, dstf, eat, w1)
    h1, p01, p11 = _pool_kernel(xp, a0, a1, batchp)
    a0, a1 = _edge_kernel(h1, srcf, dstf, eat, w2)
    h2, p02, p12 = _pool_kernel(h1, a0, a1, batchp)
    a0, a1 = _edge_kernel(h2, srcf, dstf, eat, w2)
    _, p03, p13 = _pool_kernel(h2, a0, a1, batchp)

    out = pl.pallas_call(
        _cls_body,
        out_shape=jax.ShapeDtypeStruct((B, NC_OUT), jnp.float32),
    )(p01.reshape(B, 8), p11.reshape(B, 8), p02.reshape(B, 8), p12.reshape(B, 8),
      p03.reshape(B, 8), p13.reshape(B, 8), Wl, bl, Wc, bc)
    return out
